# baseline (speedup 1.0000x reference)
import functools

import jax
import jax.numpy as jnp
from jax import lax
from jax.experimental import pallas as pl
from jax.experimental.pallas import tpu as pltpu
from jax.experimental.pallas import tpu_sc as plsc

_NSUB = 16
_BATCH = 80
_NPASS = 5


def _sc_gin_aggregate(x_user, x_item, sui, dui, siu, diu):
    n_user, d = x_user.shape
    n_item, _ = x_item.shape
    nsub, npass, cpp, b = sui.shape
    nc = npass * cpp
    assert nsub == _NSUB and b == _BATCH
    assert n_user == n_item
    rps = (n_user // nsub) // 8 * 8
    rps_last = n_user - rps * (nsub - 1)
    assert rps % 8 == 0 and rps_last % 8 == 0

    mesh = plsc.VectorSubcoreMesh(
        core_axis_name="c", subcore_axis_name="s", num_cores=2
    )

    @functools.partial(
        pl.kernel,
        out_type=[
            jax.ShapeDtypeStruct((n_item, d), jnp.float32),
            jax.ShapeDtypeStruct((n_user, d), jnp.float32),
        ],
        mesh=mesh,
        scratch_types=[
            pltpu.VMEM((cpp, b), jnp.int32),
            pltpu.VMEM((cpp, b), jnp.int32),
            pltpu.VMEM((b, d), jnp.float32),
            pltpu.VMEM((b, d), jnp.float32),
            pltpu.VMEM_SHARED((n_item, d), jnp.float32),
            pltpu.SemaphoreType.DMA,
            pltpu.SemaphoreType.DMA,
            pltpu.SemaphoreType.DMA,
            pltpu.SemaphoreType.DMA,
        ],
    )
    def agg_kernel(
        xu_hbm, xi_hbm, sui_hbm, dui_hbm, siu_hbm, diu_hbm,
        oi_hbm, ou_hbm, sidx_v, didx_v, rows0_v, rows1_v, acc_sh,
        gsem0, gsem1, ssem0, ssem1,
    ):
        c = lax.axis_index("c")
        s = lax.axis_index("s")
        rows_main = pl.ds(s * rps, rps)
        rows_last = pl.ds((nsub - 1) * rps, rps_last)

        def slab_copy(src, dst):
            @pl.when(s < nsub - 1)
            def _():
                pltpu.sync_copy(src.at[rows_main], dst.at[rows_main])

            @pl.when(s == nsub - 1)
            def _():
                pltpu.sync_copy(src.at[rows_last], dst.at[rows_last])

        def run(x_src_hbm, x_dst_hbm, s_hbm, d_hbm, o_hbm):
            slab_copy(x_dst_hbm, acc_sh)
            plsc.subcore_barrier()

            def gather_start(i, buf, sem):
                pltpu.async_copy(x_src_hbm.at[sidx_v.at[i]], buf, sem)

            def gather_wait(i, buf, sem):
                pltpu.make_async_copy(x_src_hbm.at[sidx_v.at[i]], buf, sem).wait()

            def scatter_start(i, buf, sem):
                pltpu.async_copy(buf, acc_sh.at[didx_v.at[i]], sem, add=True)

            def scatter_wait(i, buf, sem):
                pltpu.make_async_copy(buf, acc_sh.at[didx_v.at[i]], sem).wait()

            @pl.loop(0, npass)
            def _(p):
                pltpu.sync_copy(s_hbm.at[s, p], sidx_v)
                pltpu.sync_copy(d_hbm.at[s, p], didx_v)
                gather_start(0, rows0_v, gsem0)
                gather_start(1, rows1_v, gsem1)

                @pl.loop(0, cpp // 2 - 1)
                def _(k):
                    i = 2 * k
                    gather_wait(i, rows0_v, gsem0)
                    scatter_start(i, rows0_v, ssem0)
                    gather_wait(i + 1, rows1_v, gsem1)
                    scatter_start(i + 1, rows1_v, ssem1)
                    scatter_wait(i, rows0_v, ssem0)
                    gather_start(i + 2, rows0_v, gsem0)
                    scatter_wait(i + 1, rows1_v, ssem1)
                    gather_start(i + 3, rows1_v, gsem1)

                i = cpp - 3
                gather_wait(i, rows0_v, gsem0)
                scatter_start(i, rows0_v, ssem0)
                gather_wait(i + 1, rows1_v, gsem1)
                scatter_start(i + 1, rows1_v, ssem1)
                scatter_wait(i, rows0_v, ssem0)
                gather_start(i + 2, rows0_v, gsem0)
                scatter_wait(i + 1, rows1_v, ssem1)
                gather_wait(i + 2, rows0_v, gsem0)
                scatter_start(i + 2, rows0_v, ssem0)
                scatter_wait(i + 2, rows0_v, ssem0)

            plsc.subcore_barrier()
            slab_copy(acc_sh, o_hbm)

        @pl.when(c == 0)
        def _():
            run(xu_hbm, xi_hbm, sui_hbm, dui_hbm, oi_hbm)

        @pl.when(c == 1)
        def _():
            run(xi_hbm, xu_hbm, siu_hbm, diu_hbm, ou_hbm)

    return agg_kernel(x_user, x_item, sui, dui, siu, diu)


def _mlp_body(
    hi_ref, hu_ref, w1ui_ref, b1ui_ref, w2ui_ref, b2ui_ref,
    w1iu_ref, b1iu_ref, w2iu_ref, b2iu_ref, oi_ref, ou_ref,
):
    hp = jax.lax.Precision.HIGHEST
    hi = hi_ref[...]
    t = jnp.dot(hi, w1ui_ref[...], precision=hp) + b1ui_ref[...]
    t = jnp.maximum(t, 0.0)
    oi_ref[...] = jnp.dot(t, w2ui_ref[...], precision=hp) + b2ui_ref[...]
    hu = hu_ref[...]
    u = jnp.dot(hu, w1iu_ref[...], precision=hp) + b1iu_ref[...]
    u = jnp.maximum(u, 0.0)
    ou_ref[...] = jnp.dot(u, w2iu_ref[...], precision=hp) + b2iu_ref[...]


def _tc_mlps(hin_item, hin_user, w1ui, b1ui, w2ui, b2ui, w1iu, b1iu, w2iu, b2iu):
    n, d = hin_item.shape
    br = 2000
    assert n % br == 0
    spec_h = pl.BlockSpec((br, d), lambda i: (i, 0))
    spec_w = pl.BlockSpec((d, d), lambda i: (0, 0))
    spec_b = pl.BlockSpec((1, d), lambda i: (0, 0))
    return pl.pallas_call(
        _mlp_body,
        grid=(n // br,),
        in_specs=[spec_h, spec_h, spec_w, spec_b, spec_w, spec_b,
                  spec_w, spec_b, spec_w, spec_b],
        out_specs=[spec_h, spec_h],
        out_shape=[jax.ShapeDtypeStruct((n, d), jnp.float32)] * 2,
    )(
        hin_item, hin_user,
        w1ui, b1ui.reshape(1, d), w2ui, b2ui.reshape(1, d),
        w1iu, b1iu.reshape(1, d), w2iu, b2iu.reshape(1, d),
    )


def kernel(
    x_user, x_item, edge_index_user_item, edge_index_item_user,
    W1_ui, b1_ui, W2_ui, b2_ui, W1_iu, b1_iu, W2_iu, b2_iu,
):
    e = edge_index_user_item.shape[1]
    assert e % (_NSUB * _BATCH) == 0
    nc = e // (_NSUB * _BATCH)
    assert nc % _NPASS == 0
    cpp = nc // _NPASS
    ei_ui = edge_index_user_item.astype(jnp.int32)
    ei_iu = edge_index_item_user.astype(jnp.int32)
    sui = ei_ui[0].reshape(_NSUB, _NPASS, cpp, _BATCH)
    dui = ei_ui[1].reshape(_NSUB, _NPASS, cpp, _BATCH)
    siu = ei_iu[0].reshape(_NSUB, _NPASS, cpp, _BATCH)
    diu = ei_iu[1].reshape(_NSUB, _NPASS, cpp, _BATCH)

    hin_item, hin_user = _sc_gin_aggregate(x_user, x_item, sui, dui, siu, diu)
    h_item, h_user = _tc_mlps(
        hin_item, hin_user,
        W1_ui, b1_ui, W2_ui, b2_ui, W1_iu, b1_iu, W2_iu, b2_iu,
    )
    return (h_user, h_item)


# speedup vs baseline: 1.1208x; 1.1208x over previous
import functools

import jax
import jax.numpy as jnp
from jax import lax
from jax.experimental import pallas as pl
from jax.experimental.pallas import tpu as pltpu
from jax.experimental.pallas import tpu_sc as plsc

_NSUB = 16
_BATCH = 80
_NPASS = 5


def _sc_gin_aggregate(x_user, x_item, idx):
    n_user, d = x_user.shape
    n_item, _ = x_item.shape
    _, nsub, npass, cpp, b = idx.shape
    assert nsub == _NSUB and b == _BATCH
    assert n_user == n_item
    rps = (n_user // nsub) // 8 * 8
    rps_last = n_user - rps * (nsub - 1)
    assert rps % 8 == 0 and rps_last % 8 == 0

    mesh = plsc.VectorSubcoreMesh(
        core_axis_name="c", subcore_axis_name="s", num_cores=2
    )

    @functools.partial(
        pl.kernel,
        out_type=[
            jax.ShapeDtypeStruct((n_item, d), jnp.float32),
            jax.ShapeDtypeStruct((n_user, d), jnp.float32),
        ],
        mesh=mesh,
        scratch_types=[
            pltpu.VMEM((cpp, b), jnp.int32),
            pltpu.VMEM((cpp, b), jnp.int32),
            pltpu.VMEM((b, d), jnp.float32),
            pltpu.VMEM((b, d), jnp.float32),
            pltpu.VMEM_SHARED((n_item, d), jnp.float32),
            pltpu.SemaphoreType.DMA,
            pltpu.SemaphoreType.DMA,
        ],
    )
    def agg_kernel(
        xu_hbm, xi_hbm, idx_hbm,
        oi_hbm, ou_hbm, sidx_v, didx_v, rows0_v, rows1_v, acc_sh,
        gsem0, gsem1,
    ):
        c = lax.axis_index("c")
        s = lax.axis_index("s")
        rows_main = pl.ds(s * rps, rps)
        rows_last = pl.ds((nsub - 1) * rps, rps_last)

        def slab_copy(src, dst):
            @pl.when(s < nsub - 1)
            def _():
                pltpu.sync_copy(src.at[rows_main], dst.at[rows_main])

            @pl.when(s == nsub - 1)
            def _():
                pltpu.sync_copy(src.at[rows_last], dst.at[rows_last])

        def run(x_src_hbm, x_dst_hbm, si, di, o_hbm):
            slab_copy(x_dst_hbm, acc_sh)
            plsc.subcore_barrier()

            def gather_start(i, buf, sem):
                pltpu.async_copy(x_src_hbm.at[sidx_v.at[i]], buf, sem)

            def gather_wait(i, buf, sem):
                pltpu.make_async_copy(x_src_hbm.at[sidx_v.at[i]], buf, sem).wait()

            def scatter_add(i, buf):
                pltpu.sync_copy(buf, acc_sh.at[didx_v.at[i]], add=True)

            @pl.loop(0, npass)
            def _(p):
                pltpu.sync_copy(idx_hbm.at[si, s, p], sidx_v)
                pltpu.sync_copy(idx_hbm.at[di, s, p], didx_v)
                gather_start(0, rows0_v, gsem0)

                @pl.loop(0, cpp // 2)
                def _(k):
                    i = 2 * k
                    gather_start(i + 1, rows1_v, gsem1)
                    gather_wait(i, rows0_v, gsem0)
                    scatter_add(i, rows0_v)
                    gather_start(i + 2, rows0_v, gsem0)
                    gather_wait(i + 1, rows1_v, gsem1)
                    scatter_add(i + 1, rows1_v)

                gather_wait(cpp - 1, rows0_v, gsem0)
                scatter_add(cpp - 1, rows0_v)

            plsc.subcore_barrier()
            slab_copy(acc_sh, o_hbm)

        @pl.when(c == 0)
        def _():
            run(xu_hbm, xi_hbm, 0, 1, oi_hbm)

        @pl.when(c == 1)
        def _():
            run(xi_hbm, xu_hbm, 2, 3, ou_hbm)

    return agg_kernel(x_user, x_item, idx)


def _mlp_body(
    hi_ref, hu_ref, w1ui_ref, b1ui_ref, w2ui_ref, b2ui_ref,
    w1iu_ref, b1iu_ref, w2iu_ref, b2iu_ref, oi_ref, ou_ref,
):
    hp = jax.lax.Precision.HIGHEST
    hi = hi_ref[...]
    t = jnp.dot(hi, w1ui_ref[...], precision=hp) + b1ui_ref[...]
    t = jnp.maximum(t, 0.0)
    oi_ref[...] = jnp.dot(t, w2ui_ref[...], precision=hp) + b2ui_ref[...]
    hu = hu_ref[...]
    u = jnp.dot(hu, w1iu_ref[...], precision=hp) + b1iu_ref[...]
    u = jnp.maximum(u, 0.0)
    ou_ref[...] = jnp.dot(u, w2iu_ref[...], precision=hp) + b2iu_ref[...]


def _tc_mlps(hin_item, hin_user, w1ui, b1ui, w2ui, b2ui, w1iu, b1iu, w2iu, b2iu):
    n, d = hin_item.shape
    br = 1000
    assert n % br == 0
    spec_h = pl.BlockSpec((br, d), lambda i: (i, 0))
    spec_w = pl.BlockSpec((d, d), lambda i: (0, 0))
    spec_b = pl.BlockSpec((1, d), lambda i: (0, 0))
    return pl.pallas_call(
        _mlp_body,
        grid=(n // br,),
        in_specs=[spec_h, spec_h, spec_w, spec_b, spec_w, spec_b,
                  spec_w, spec_b, spec_w, spec_b],
        out_specs=[spec_h, spec_h],
        out_shape=[jax.ShapeDtypeStruct((n, d), jnp.float32)] * 2,
    )(
        hin_item, hin_user,
        w1ui, b1ui.reshape(1, d), w2ui, b2ui.reshape(1, d),
        w1iu, b1iu.reshape(1, d), w2iu, b2iu.reshape(1, d),
    )


def kernel(
    x_user, x_item, edge_index_user_item, edge_index_item_user,
    W1_ui, b1_ui, W2_ui, b2_ui, W1_iu, b1_iu, W2_iu, b2_iu,
):
    e = edge_index_user_item.shape[1]
    assert e % (_NSUB * _BATCH) == 0
    nc = e // (_NSUB * _BATCH)
    assert nc % _NPASS == 0
    cpp = nc // _NPASS
    idx = jnp.concatenate(
        [edge_index_user_item, edge_index_item_user]
    ).astype(jnp.int32).reshape(4, _NSUB, _NPASS, cpp, _BATCH)

    hin_item, hin_user = _sc_gin_aggregate(x_user, x_item, idx)
    h_item, h_user = _tc_mlps(
        hin_item, hin_user,
        W1_ui, b1_ui, W2_ui, b2_ui, W1_iu, b1_iu, W2_iu, b2_iu,
    )
    return (h_user, h_item)


# speedup vs baseline: 1.1249x; 1.0037x over previous
import functools

import jax
import jax.numpy as jnp
from jax import lax
from jax.experimental import pallas as pl
from jax.experimental.pallas import tpu as pltpu
from jax.experimental.pallas import tpu_sc as plsc

_NSUB = 16
_BATCH = 80
_NPASS = 5


def _sc_gin_aggregate(x_user, x_item, idx):
    n_user, d = x_user.shape
    n_item, _ = x_item.shape
    _, nsub, npass, cpp, b = idx.shape
    assert nsub == _NSUB and b == _BATCH
    assert n_user == n_item
    rps = (n_user // nsub) // 8 * 8
    rps_last = n_user - rps * (nsub - 1)
    assert rps % 8 == 0 and rps_last % 8 == 0

    mesh = plsc.VectorSubcoreMesh(
        core_axis_name="c", subcore_axis_name="s", num_cores=2
    )

    @functools.partial(
        pl.kernel,
        out_type=[
            jax.ShapeDtypeStruct((n_item, d), jnp.float32),
            jax.ShapeDtypeStruct((n_user, d), jnp.float32),
        ],
        mesh=mesh,
        scratch_types=[
            pltpu.VMEM((cpp, b), jnp.int32),
            pltpu.VMEM((cpp, b), jnp.int32),
            pltpu.VMEM((b, d), jnp.float32),
            pltpu.VMEM((b, d), jnp.float32),
            pltpu.VMEM_SHARED((n_item, d), jnp.float32),
            pltpu.SemaphoreType.DMA,
            pltpu.SemaphoreType.DMA,
        ],
    )
    def agg_kernel(
        xu_hbm, xi_hbm, idx_hbm,
        oi_hbm, ou_hbm, sidx_v, didx_v, rows0_v, rows1_v, acc_sh,
        gsem0, gsem1,
    ):
        c = lax.axis_index("c")
        s = lax.axis_index("s")
        rows_main = pl.ds(s * rps, rps)
        rows_last = pl.ds((nsub - 1) * rps, rps_last)

        def slab_copy(src, dst):
            @pl.when(s < nsub - 1)
            def _():
                pltpu.sync_copy(src.at[rows_main], dst.at[rows_main])

            @pl.when(s == nsub - 1)
            def _():
                pltpu.sync_copy(src.at[rows_last], dst.at[rows_last])

        def run(x_src_hbm, x_dst_hbm, si, di, o_hbm):
            slab_copy(x_dst_hbm, acc_sh)
            plsc.subcore_barrier()

            def gather_start(i, buf, sem):
                pltpu.async_copy(x_src_hbm.at[sidx_v.at[i]], buf, sem)

            def gather_wait(i, buf, sem):
                pltpu.make_async_copy(x_src_hbm.at[sidx_v.at[i]], buf, sem).wait()

            def scatter_add(i, buf):
                pltpu.sync_copy(buf, acc_sh.at[didx_v.at[i]], add=True)

            @pl.loop(0, npass)
            def _(p):
                pltpu.sync_copy(idx_hbm.at[si, s, p], sidx_v)
                pltpu.sync_copy(idx_hbm.at[di, s, p], didx_v)
                gather_start(0, rows0_v, gsem0)

                @pl.loop(0, cpp // 2)
                def _(k):
                    i = 2 * k
                    gather_start(i + 1, rows1_v, gsem1)
                    gather_wait(i, rows0_v, gsem0)
                    scatter_add(i, rows0_v)
                    gather_start(i + 2, rows0_v, gsem0)
                    gather_wait(i + 1, rows1_v, gsem1)
                    scatter_add(i + 1, rows1_v)

                gather_wait(cpp - 1, rows0_v, gsem0)
                scatter_add(cpp - 1, rows0_v)

            plsc.subcore_barrier()
            slab_copy(acc_sh, o_hbm)

        @pl.when(c == 0)
        def _():
            run(xu_hbm, xi_hbm, 0, 1, oi_hbm)

        @pl.when(c == 1)
        def _():
            run(xi_hbm, xu_hbm, 2, 3, ou_hbm)

    return agg_kernel(x_user, x_item, idx)


def _mlp_body(
    hi_ref, hu_ref, w1ui_ref, b1ui_ref, w2ui_ref, b2ui_ref,
    w1iu_ref, b1iu_ref, w2iu_ref, b2iu_ref, oi_ref, ou_ref,
):
    hp = jax.lax.Precision.HIGHEST
    hi = hi_ref[...]
    t = jnp.dot(hi, w1ui_ref[...], precision=hp) + b1ui_ref[...]
    t = jnp.maximum(t, 0.0)
    oi_ref[...] = jnp.dot(t, w2ui_ref[...], precision=hp) + b2ui_ref[...]
    hu = hu_ref[...]
    u = jnp.dot(hu, w1iu_ref[...], precision=hp) + b1iu_ref[...]
    u = jnp.maximum(u, 0.0)
    ou_ref[...] = jnp.dot(u, w2iu_ref[...], precision=hp) + b2iu_ref[...]


def _tc_mlps(hin_item, hin_user, w1ui, b1ui, w2ui, b2ui, w1iu, b1iu, w2iu, b2iu):
    n, d = hin_item.shape
    br = 5000
    assert n % br == 0
    spec_h = pl.BlockSpec((br, d), lambda i: (i, 0))
    spec_w = pl.BlockSpec((d, d), lambda i: (0, 0))
    spec_b = pl.BlockSpec((1, d), lambda i: (0, 0))
    return pl.pallas_call(
        _mlp_body,
        grid=(n // br,),
        in_specs=[spec_h, spec_h, spec_w, spec_b, spec_w, spec_b,
                  spec_w, spec_b, spec_w, spec_b],
        out_specs=[spec_h, spec_h],
        out_shape=[jax.ShapeDtypeStruct((n, d), jnp.float32)] * 2,
    )(
        hin_item, hin_user,
        w1ui, b1ui.reshape(1, d), w2ui, b2ui.reshape(1, d),
        w1iu, b1iu.reshape(1, d), w2iu, b2iu.reshape(1, d),
    )


def kernel(
    x_user, x_item, edge_index_user_item, edge_index_item_user,
    W1_ui, b1_ui, W2_ui, b2_ui, W1_iu, b1_iu, W2_iu, b2_iu,
):
    e = edge_index_user_item.shape[1]
    assert e % (_NSUB * _BATCH) == 0
    nc = e // (_NSUB * _BATCH)
    assert nc % _NPASS == 0
    cpp = nc // _NPASS
    idx = jnp.concatenate(
        [edge_index_user_item, edge_index_item_user]
    ).astype(jnp.int32).reshape(4, _NSUB, _NPASS, cpp, _BATCH)

    hin_item, hin_user = _sc_gin_aggregate(x_user, x_item, idx)
    h_item, h_user = _tc_mlps(
        hin_item, hin_user,
        W1_ui, b1_ui, W2_ui, b2_ui, W1_iu, b1_iu, W2_iu, b2_iu,
    )
    return (h_user, h_item)


# speedup vs baseline: 1.2631x; 1.1229x over previous
import functools

import jax
import jax.numpy as jnp
from jax import lax
from jax.experimental import pallas as pl
from jax.experimental.pallas import tpu as pltpu
from jax.experimental.pallas import tpu_sc as plsc

_NSUB = 16
_BATCH = 80
_NPASS = 5


def _sc_gin_aggregate(x_user, x_item, idx):
    n_user, d = x_user.shape
    n_item, _ = x_item.shape
    _, nsub, npass, cpp, b = idx.shape
    assert nsub == _NSUB and b == _BATCH
    assert n_user == n_item
    rps = (n_user // nsub) // 8 * 8
    rps_last = n_user - rps * (nsub - 1)
    assert rps % 8 == 0 and rps_last % 8 == 0

    mesh = plsc.VectorSubcoreMesh(
        core_axis_name="c", subcore_axis_name="s", num_cores=2
    )

    @functools.partial(
        pl.kernel,
        out_type=[
            jax.ShapeDtypeStruct((n_item, d), jnp.float32),
            jax.ShapeDtypeStruct((n_user, d), jnp.float32),
        ],
        mesh=mesh,
        scratch_types=[
            pltpu.VMEM((cpp, b), jnp.int32),
            pltpu.VMEM((cpp, b), jnp.int32),
            pltpu.VMEM((b, d), jnp.float32),
            pltpu.VMEM((b, d), jnp.float32),
            pltpu.VMEM_SHARED((n_item, d), jnp.float32),
            pltpu.SemaphoreType.DMA,
            pltpu.SemaphoreType.DMA,
        ],
    )
    def agg_kernel(
        xu_hbm, xi_hbm, idx_hbm,
        oi_hbm, ou_hbm, sidx_v, didx_v, rows0_v, rows1_v, acc_sh,
        gsem0, gsem1,
    ):
        c = lax.axis_index("c")
        s = lax.axis_index("s")
        rows_main = pl.ds(s * rps, rps)
        rows_last = pl.ds((nsub - 1) * rps, rps_last)

        def slab_copy(src, dst):
            @pl.when(s < nsub - 1)
            def _():
                pltpu.sync_copy(src.at[rows_main], dst.at[rows_main])

            @pl.when(s == nsub - 1)
            def _():
                pltpu.sync_copy(src.at[rows_last], dst.at[rows_last])

        def run(x_src_hbm, x_dst_hbm, si, di, o_hbm):
            slab_copy(x_dst_hbm, acc_sh)
            plsc.subcore_barrier()

            def gather_start(i, buf, sem):
                pltpu.async_copy(x_src_hbm.at[sidx_v.at[i]], buf, sem)

            def gather_wait(i, buf, sem):
                pltpu.make_async_copy(x_src_hbm.at[sidx_v.at[i]], buf, sem).wait()

            def scatter_add(i, buf):
                pltpu.sync_copy(buf, acc_sh.at[didx_v.at[i]], add=True)

            @pl.loop(0, npass)
            def _(p):
                pltpu.sync_copy(idx_hbm.at[si, s, p], sidx_v)
                pltpu.sync_copy(idx_hbm.at[di, s, p], didx_v)
                gather_start(0, rows0_v, gsem0)

                @pl.loop(0, cpp // 2)
                def _(k):
                    i = 2 * k
                    gather_start(i + 1, rows1_v, gsem1)
                    gather_wait(i, rows0_v, gsem0)
                    scatter_add(i, rows0_v)
                    gather_start(i + 2, rows0_v, gsem0)
                    gather_wait(i + 1, rows1_v, gsem1)
                    scatter_add(i + 1, rows1_v)

                gather_wait(cpp - 1, rows0_v, gsem0)
                scatter_add(cpp - 1, rows0_v)

            plsc.subcore_barrier()
            slab_copy(acc_sh, o_hbm)

        @pl.when(c == 0)
        def _():
            run(xu_hbm, xi_hbm, 0, 1, oi_hbm)

        @pl.when(c == 1)
        def _():
            run(xi_hbm, xu_hbm, 2, 3, ou_hbm)

    return agg_kernel(x_user, x_item, idx)


def _mlp_body(
    hi_ref, hu_ref, w1ui_ref, b1ui_ref, w2ui_ref, b2ui_ref,
    w1iu_ref, b1iu_ref, w2iu_ref, b2iu_ref, oi_ref, ou_ref,
):
    hp = jax.lax.Precision.HIGHEST
    hi = hi_ref[...]
    t = jnp.dot(hi, w1ui_ref[...], precision=hp) + b1ui_ref[...]
    t = jnp.maximum(t, 0.0)
    oi_ref[...] = jnp.dot(t, w2ui_ref[...], precision=hp) + b2ui_ref[...]
    hu = hu_ref[...]
    u = jnp.dot(hu, w1iu_ref[...], precision=hp) + b1iu_ref[...]
    u = jnp.maximum(u, 0.0)
    ou_ref[...] = jnp.dot(u, w2iu_ref[...], precision=hp) + b2iu_ref[...]


def _tc_mlps(hin_item, hin_user, w1ui, b1ui, w2ui, b2ui, w1iu, b1iu, w2iu, b2iu):
    n, d = hin_item.shape
    br = 2000
    assert n % br == 0
    spec_h = pl.BlockSpec((br, d), lambda i: (i, 0))
    spec_w = pl.BlockSpec((d, d), lambda i: (0, 0))
    spec_b = pl.BlockSpec((1, d), lambda i: (0, 0))
    return pl.pallas_call(
        _mlp_body,
        grid=(n // br,),
        in_specs=[spec_h, spec_h, spec_w, spec_b, spec_w, spec_b,
                  spec_w, spec_b, spec_w, spec_b],
        out_specs=[spec_h, spec_h],
        out_shape=[jax.ShapeDtypeStruct((n, d), jnp.float32)] * 2,
    )(
        hin_item, hin_user,
        w1ui, b1ui.reshape(1, d), w2ui, b2ui.reshape(1, d),
        w1iu, b1iu.reshape(1, d), w2iu, b2iu.reshape(1, d),
    )


def kernel(
    x_user, x_item, edge_index_user_item, edge_index_item_user,
    W1_ui, b1_ui, W2_ui, b2_ui, W1_iu, b1_iu, W2_iu, b2_iu,
):
    e = edge_index_user_item.shape[1]
    assert e % (_NSUB * _BATCH) == 0
    nc = e // (_NSUB * _BATCH)
    assert nc % _NPASS == 0
    cpp = nc // _NPASS
    idx = jnp.concatenate(
        [edge_index_user_item, edge_index_item_user]
    ).astype(jnp.int32).reshape(4, _NSUB, _NPASS, cpp, _BATCH)

    hin_item, hin_user = _sc_gin_aggregate(x_user, x_item, idx)
    h_item, h_user = _tc_mlps(
        hin_item, hin_user,
        W1_ui, b1_ui, W2_ui, b2_ui, W1_iu, b1_iu, W2_iu, b2_iu,
    )
    return (h_user, h_item)


# speedup vs baseline: 1.4106x; 1.1167x over previous
import functools

import jax
import jax.numpy as jnp
from jax import lax
from jax.experimental import pallas as pl
from jax.experimental.pallas import tpu as pltpu
from jax.experimental.pallas import tpu_sc as plsc

_NSUB = 16
_BATCH = 80
_NPASS = 5


def _sc_gin_aggregate(x_user, x_item, idx):
    n_user, d = x_user.shape
    n_item, _ = x_item.shape
    _, nsub, npass, cpp, b = idx.shape
    assert nsub == _NSUB and b == _BATCH
    assert cpp % 3 == 1 and cpp >= 4
    assert n_user == n_item
    rps = (n_user // nsub) // 8 * 8
    rps_last = n_user - rps * (nsub - 1)
    assert rps % 8 == 0 and rps_last % 8 == 0

    mesh = plsc.VectorSubcoreMesh(
        core_axis_name="c", subcore_axis_name="s", num_cores=2
    )

    @functools.partial(
        pl.kernel,
        out_type=[
            jax.ShapeDtypeStruct((n_item, d), jnp.float32),
            jax.ShapeDtypeStruct((n_user, d), jnp.float32),
        ],
        mesh=mesh,
        scratch_types=[
            pltpu.VMEM((cpp, b), jnp.int32),
            pltpu.VMEM((cpp, b), jnp.int32),
            pltpu.VMEM((b, d), jnp.float32),
            pltpu.VMEM((b, d), jnp.float32),
            pltpu.VMEM((b, d), jnp.float32),
            pltpu.VMEM_SHARED((n_item, d), jnp.float32),
            pltpu.SemaphoreType.DMA,
            pltpu.SemaphoreType.DMA,
            pltpu.SemaphoreType.DMA,
        ],
    )
    def agg_kernel(
        xu_hbm, xi_hbm, idx_hbm,
        oi_hbm, ou_hbm, sidx_v, didx_v, rows0_v, rows1_v, rows2_v, acc_sh,
        gsem0, gsem1, gsem2,
    ):
        c = lax.axis_index("c")
        s = lax.axis_index("s")
        rows_main = pl.ds(s * rps, rps)
        rows_last = pl.ds((nsub - 1) * rps, rps_last)

        def slab_copy(src, dst):
            @pl.when(s < nsub - 1)
            def _():
                pltpu.sync_copy(src.at[rows_main], dst.at[rows_main])

            @pl.when(s == nsub - 1)
            def _():
                pltpu.sync_copy(src.at[rows_last], dst.at[rows_last])

        def run(x_src_hbm, x_dst_hbm, si, di, o_hbm):
            slab_copy(x_dst_hbm, acc_sh)
            plsc.subcore_barrier()

            def gather_start(i, buf, sem):
                pltpu.async_copy(x_src_hbm.at[sidx_v.at[i]], buf, sem)

            def gather_wait(i, buf, sem):
                pltpu.make_async_copy(x_src_hbm.at[sidx_v.at[i]], buf, sem).wait()

            def scatter_add(i, buf):
                pltpu.sync_copy(buf, acc_sh.at[didx_v.at[i]], add=True)

            @pl.loop(0, npass)
            def _(p):
                pltpu.sync_copy(idx_hbm.at[si, s, p], sidx_v)
                pltpu.sync_copy(idx_hbm.at[di, s, p], didx_v)
                gather_start(0, rows0_v, gsem0)
                gather_start(1, rows1_v, gsem1)
                gather_start(2, rows2_v, gsem2)

                @pl.loop(0, cpp // 3 - 1)
                def _(k):
                    i = 3 * k
                    gather_wait(i, rows0_v, gsem0)
                    scatter_add(i, rows0_v)
                    gather_start(i + 3, rows0_v, gsem0)
                    gather_wait(i + 1, rows1_v, gsem1)
                    scatter_add(i + 1, rows1_v)
                    gather_start(i + 4, rows1_v, gsem1)
                    gather_wait(i + 2, rows2_v, gsem2)
                    scatter_add(i + 2, rows2_v)
                    gather_start(i + 5, rows2_v, gsem2)

                i = cpp - 4
                gather_wait(i, rows0_v, gsem0)
                scatter_add(i, rows0_v)
                gather_start(i + 3, rows0_v, gsem0)
                gather_wait(i + 1, rows1_v, gsem1)
                scatter_add(i + 1, rows1_v)
                gather_wait(i + 2, rows2_v, gsem2)
                scatter_add(i + 2, rows2_v)
                gather_wait(i + 3, rows0_v, gsem0)
                scatter_add(i + 3, rows0_v)

            plsc.subcore_barrier()
            slab_copy(acc_sh, o_hbm)

        @pl.when(c == 0)
        def _():
            run(xu_hbm, xi_hbm, 0, 1, oi_hbm)

        @pl.when(c == 1)
        def _():
            run(xi_hbm, xu_hbm, 2, 3, ou_hbm)

    return agg_kernel(x_user, x_item, idx)


def _mlp_body(
    hi_ref, hu_ref, w1ui_ref, b1ui_ref, w2ui_ref, b2ui_ref,
    w1iu_ref, b1iu_ref, w2iu_ref, b2iu_ref, oi_ref, ou_ref,
):
    hp = jax.lax.Precision.HIGHEST
    hi = hi_ref[...]
    t = jnp.dot(hi, w1ui_ref[...], precision=hp) + b1ui_ref[...]
    t = jnp.maximum(t, 0.0)
    oi_ref[...] = jnp.dot(t, w2ui_ref[...], precision=hp) + b2ui_ref[...]
    hu = hu_ref[...]
    u = jnp.dot(hu, w1iu_ref[...], precision=hp) + b1iu_ref[...]
    u = jnp.maximum(u, 0.0)
    ou_ref[...] = jnp.dot(u, w2iu_ref[...], precision=hp) + b2iu_ref[...]


def _tc_mlps(hin_item, hin_user, w1ui, b1ui, w2ui, b2ui, w1iu, b1iu, w2iu, b2iu):
    n, d = hin_item.shape
    br = 2000
    assert n % br == 0
    spec_h = pl.BlockSpec((br, d), lambda i: (i, 0))
    spec_w = pl.BlockSpec((d, d), lambda i: (0, 0))
    spec_b = pl.BlockSpec((1, d), lambda i: (0, 0))
    return pl.pallas_call(
        _mlp_body,
        grid=(n // br,),
        in_specs=[spec_h, spec_h, spec_w, spec_b, spec_w, spec_b,
                  spec_w, spec_b, spec_w, spec_b],
        out_specs=[spec_h, spec_h],
        out_shape=[jax.ShapeDtypeStruct((n, d), jnp.float32)] * 2,
    )(
        hin_item, hin_user,
        w1ui, b1ui.reshape(1, d), w2ui, b2ui.reshape(1, d),
        w1iu, b1iu.reshape(1, d), w2iu, b2iu.reshape(1, d),
    )


def kernel(
    x_user, x_item, edge_index_user_item, edge_index_item_user,
    W1_ui, b1_ui, W2_ui, b2_ui, W1_iu, b1_iu, W2_iu, b2_iu,
):
    e = edge_index_user_item.shape[1]
    assert e % (_NSUB * _BATCH) == 0
    nc = e // (_NSUB * _BATCH)
    assert nc % _NPASS == 0
    cpp = nc // _NPASS
    idx = jnp.concatenate(
        [edge_index_user_item, edge_index_item_user]
    ).astype(jnp.int32).reshape(4, _NSUB, _NPASS, cpp, _BATCH)

    hin_item, hin_user = _sc_gin_aggregate(x_user, x_item, idx)
    h_item, h_user = _tc_mlps(
        hin_item, hin_user,
        W1_ui, b1_ui, W2_ui, b2_ui, W1_iu, b1_iu, W2_iu, b2_iu,
    )
    return (h_user, h_item)


# speedup vs baseline: 1.5718x; 1.1143x over previous
import functools

import jax
import jax.numpy as jnp
from jax import lax
from jax.experimental import pallas as pl
from jax.experimental.pallas import tpu as pltpu
from jax.experimental.pallas import tpu_sc as plsc

_NSUB = 16
_BATCH = 80
_NPASS = 5


def _sc_gin_aggregate(x_user, x_item, idx):
    n_user, d = x_user.shape
    n_item, _ = x_item.shape
    _, nsub, npass, cpp, b = idx.shape
    assert nsub == _NSUB and b == _BATCH
    assert cpp % 3 == 1 and cpp >= 4
    assert n_user == n_item
    rps = (n_user // nsub) // 8 * 8
    rps_last = n_user - rps * (nsub - 1)
    assert rps % 8 == 0 and rps_last % 8 == 0

    mesh = plsc.VectorSubcoreMesh(
        core_axis_name="c", subcore_axis_name="s", num_cores=2
    )

    @functools.partial(
        pl.kernel,
        out_type=[
            jax.ShapeDtypeStruct((n_item, d), jnp.float32),
            jax.ShapeDtypeStruct((n_user, d), jnp.float32),
        ],
        mesh=mesh,
        scratch_types=[
            pltpu.VMEM((cpp, b), jnp.int32),
            pltpu.VMEM((cpp, b), jnp.int32),
            pltpu.VMEM((b, d), jnp.float32),
            pltpu.VMEM((b, d), jnp.float32),
            pltpu.VMEM((b, d), jnp.float32),
            pltpu.VMEM_SHARED((n_item, d), jnp.float32),
            pltpu.SemaphoreType.DMA,
            pltpu.SemaphoreType.DMA,
            pltpu.SemaphoreType.DMA,
        ],
    )
    def agg_kernel(
        xu_hbm, xi_hbm, idx_hbm,
        oi_hbm, ou_hbm, sidx_v, didx_v, rows0_v, rows1_v, rows2_v, acc_sh,
        gsem0, gsem1, gsem2,
    ):
        c = lax.axis_index("c")
        s = lax.axis_index("s")
        rows_main = pl.ds(s * rps, rps)
        rows_last = pl.ds((nsub - 1) * rps, rps_last)

        def slab_copy(src, dst):
            @pl.when(s < nsub - 1)
            def _():
                pltpu.sync_copy(src.at[rows_main], dst.at[rows_main])

            @pl.when(s == nsub - 1)
            def _():
                pltpu.sync_copy(src.at[rows_last], dst.at[rows_last])

        def run(x_src_hbm, x_dst_hbm, si, di, o_hbm):
            slab_copy(x_dst_hbm, acc_sh)
            plsc.subcore_barrier()

            def gather_start(i, buf, sem):
                pltpu.async_copy(x_src_hbm.at[sidx_v.at[i]], buf, sem)

            def gather_wait(i, buf, sem):
                pltpu.make_async_copy(x_src_hbm.at[sidx_v.at[i]], buf, sem).wait()

            def scatter_add(i, buf):
                pltpu.sync_copy(buf, acc_sh.at[didx_v.at[i]], add=True)

            @pl.loop(0, npass)
            def _(p):
                pltpu.sync_copy(idx_hbm.at[si, s, p], sidx_v)
                pltpu.sync_copy(idx_hbm.at[di, s, p], didx_v)
                gather_start(0, rows0_v, gsem0)
                gather_start(1, rows1_v, gsem1)
                gather_start(2, rows2_v, gsem2)

                @pl.loop(0, cpp // 3 - 1)
                def _(k):
                    i = 3 * k
                    gather_wait(i, rows0_v, gsem0)
                    scatter_add(i, rows0_v)
                    gather_start(i + 3, rows0_v, gsem0)
                    gather_wait(i + 1, rows1_v, gsem1)
                    scatter_add(i + 1, rows1_v)
                    gather_start(i + 4, rows1_v, gsem1)
                    gather_wait(i + 2, rows2_v, gsem2)
                    scatter_add(i + 2, rows2_v)
                    gather_start(i + 5, rows2_v, gsem2)

                i = cpp - 4
                gather_wait(i, rows0_v, gsem0)
                scatter_add(i, rows0_v)
                gather_start(i + 3, rows0_v, gsem0)
                gather_wait(i + 1, rows1_v, gsem1)
                scatter_add(i + 1, rows1_v)
                gather_wait(i + 2, rows2_v, gsem2)
                scatter_add(i + 2, rows2_v)
                gather_wait(i + 3, rows0_v, gsem0)
                scatter_add(i + 3, rows0_v)

            plsc.subcore_barrier()
            slab_copy(acc_sh, o_hbm)

        @pl.when(c == 0)
        def _():
            run(xu_hbm, xi_hbm, 0, 1, oi_hbm)

        @pl.when(c == 1)
        def _():
            run(xi_hbm, xu_hbm, 2, 3, ou_hbm)

    return agg_kernel(x_user, x_item, idx)


def _mlp_body(
    hi_ref, hu_ref, w1ui_ref, b1ui_ref, w2ui_ref, b2ui_ref,
    w1iu_ref, b1iu_ref, w2iu_ref, b2iu_ref, oi_ref, ou_ref,
):
    hi = hi_ref[...]
    t = jnp.dot(hi, w1ui_ref[...]) + b1ui_ref[...]
    t = jnp.maximum(t, 0.0)
    oi_ref[...] = jnp.dot(t, w2ui_ref[...]) + b2ui_ref[...]
    hu = hu_ref[...]
    u = jnp.dot(hu, w1iu_ref[...]) + b1iu_ref[...]
    u = jnp.maximum(u, 0.0)
    ou_ref[...] = jnp.dot(u, w2iu_ref[...]) + b2iu_ref[...]


def _tc_mlps(hin_item, hin_user, w1ui, b1ui, w2ui, b2ui, w1iu, b1iu, w2iu, b2iu):
    n, d = hin_item.shape
    br = 2000
    assert n % br == 0
    spec_h = pl.BlockSpec((br, d), lambda i: (i, 0))
    spec_w = pl.BlockSpec((d, d), lambda i: (0, 0))
    spec_b = pl.BlockSpec((1, d), lambda i: (0, 0))
    return pl.pallas_call(
        _mlp_body,
        grid=(n // br,),
        in_specs=[spec_h, spec_h, spec_w, spec_b, spec_w, spec_b,
                  spec_w, spec_b, spec_w, spec_b],
        out_specs=[spec_h, spec_h],
        out_shape=[jax.ShapeDtypeStruct((n, d), jnp.float32)] * 2,
    )(
        hin_item, hin_user,
        w1ui, b1ui.reshape(1, d), w2ui, b2ui.reshape(1, d),
        w1iu, b1iu.reshape(1, d), w2iu, b2iu.reshape(1, d),
    )


def kernel(
    x_user, x_item, edge_index_user_item, edge_index_item_user,
    W1_ui, b1_ui, W2_ui, b2_ui, W1_iu, b1_iu, W2_iu, b2_iu,
):
    e = edge_index_user_item.shape[1]
    assert e % (_NSUB * _BATCH) == 0
    nc = e // (_NSUB * _BATCH)
    assert nc % _NPASS == 0
    cpp = nc // _NPASS
    idx = jnp.concatenate(
        [edge_index_user_item, edge_index_item_user]
    ).astype(jnp.int32).reshape(4, _NSUB, _NPASS, cpp, _BATCH)

    hin_item, hin_user = _sc_gin_aggregate(x_user, x_item, idx)
    h_item, h_user = _tc_mlps(
        hin_item, hin_user,
        W1_ui, b1_ui, W2_ui, b2_ui, W1_iu, b1_iu, W2_iu, b2_iu,
    )
    return (h_user, h_item)


# speedup vs baseline: 1.6380x; 1.0421x over previous
import functools

import jax
import jax.numpy as jnp
from jax import lax
from jax.experimental import pallas as pl
from jax.experimental.pallas import tpu as pltpu
from jax.experimental.pallas import tpu_sc as plsc

_NSUB = 16
_BATCH = 80
_NPASS = 5


def _sc_gin_aggregate(x_user, x_item, idx):
    n_user, d = x_user.shape
    n_item, _ = x_item.shape
    _, nsub, npass, cpp, b = idx.shape
    assert nsub == _NSUB and b == _BATCH
    assert cpp % 3 == 1 and cpp >= 4
    assert n_user == n_item
    rps = (n_user // nsub) // 8 * 8
    rps_last = n_user - rps * (nsub - 1)
    assert rps % 8 == 0 and rps_last % 8 == 0

    mesh = plsc.VectorSubcoreMesh(
        core_axis_name="c", subcore_axis_name="s", num_cores=2
    )

    @functools.partial(
        pl.kernel,
        out_type=[
            jax.ShapeDtypeStruct((n_item, d), jnp.float32),
            jax.ShapeDtypeStruct((n_user, d), jnp.float32),
        ],
        mesh=mesh,
        scratch_types=[
            pltpu.VMEM((2, cpp, b), jnp.int32),
            pltpu.VMEM((2, cpp, b), jnp.int32),
            pltpu.VMEM((b, d), jnp.float32),
            pltpu.VMEM((b, d), jnp.float32),
            pltpu.VMEM((b, d), jnp.float32),
            pltpu.VMEM_SHARED((n_item, d), jnp.float32),
            pltpu.SemaphoreType.DMA,
            pltpu.SemaphoreType.DMA,
            pltpu.SemaphoreType.DMA,
            pltpu.SemaphoreType.DMA,
            pltpu.SemaphoreType.DMA,
        ],
    )
    def agg_kernel(
        xu_hbm, xi_hbm, idx_hbm,
        oi_hbm, ou_hbm, idx0_v, idx1_v, rows0_v, rows1_v, rows2_v, acc_sh,
        gsem0, gsem1, gsem2, isem0, isem1,
    ):
        c = lax.axis_index("c")
        s = lax.axis_index("s")
        rows_main = pl.ds(s * rps, rps)
        rows_last = pl.ds((nsub - 1) * rps, rps_last)

        def slab_copy(src, dst):
            @pl.when(s < nsub - 1)
            def _():
                pltpu.sync_copy(src.at[rows_main], dst.at[rows_main])

            @pl.when(s == nsub - 1)
            def _():
                pltpu.sync_copy(src.at[rows_last], dst.at[rows_last])

        def run(x_src_hbm, x_dst_hbm, si, o_hbm):
            def stage_start(p, ibuf, isem):
                pltpu.async_copy(idx_hbm.at[pl.ds(si, 2), s, p], ibuf, isem)

            def stage_wait(p, ibuf, isem):
                pltpu.make_async_copy(
                    idx_hbm.at[pl.ds(si, 2), s, p], ibuf, isem).wait()

            stage_start(0, idx0_v, isem0)
            slab_copy(x_dst_hbm, acc_sh)
            plsc.subcore_barrier()

            def gather_start(sidx, i, buf, sem):
                pltpu.async_copy(x_src_hbm.at[sidx.at[i]], buf, sem)

            def gather_wait(sidx, i, buf, sem):
                pltpu.make_async_copy(x_src_hbm.at[sidx.at[i]], buf, sem).wait()

            def scatter_add(didx, i, buf):
                pltpu.sync_copy(buf, acc_sh.at[didx.at[i]], add=True)

            for p in range(npass):
                ibuf, isem = (idx0_v, isem0) if p % 2 == 0 else (idx1_v, isem1)
                stage_wait(p, ibuf, isem)
                if p + 1 < npass:
                    nbuf, nsem = (idx0_v, isem0) if (p + 1) % 2 == 0 else (idx1_v, isem1)
                    stage_start(p + 1, nbuf, nsem)
                sidx = ibuf.at[0]
                didx = ibuf.at[1]
                gather_start(sidx, 0, rows0_v, gsem0)
                gather_start(sidx, 1, rows1_v, gsem1)
                gather_start(sidx, 2, rows2_v, gsem2)

                @pl.loop(0, cpp // 3 - 1)
                def _(k):
                    i = 3 * k
                    gather_wait(sidx, i, rows0_v, gsem0)
                    scatter_add(didx, i, rows0_v)
                    gather_start(sidx, i + 3, rows0_v, gsem0)
                    gather_wait(sidx, i + 1, rows1_v, gsem1)
                    scatter_add(didx, i + 1, rows1_v)
                    gather_start(sidx, i + 4, rows1_v, gsem1)
                    gather_wait(sidx, i + 2, rows2_v, gsem2)
                    scatter_add(didx, i + 2, rows2_v)
                    gather_start(sidx, i + 5, rows2_v, gsem2)

                i = cpp - 4
                gather_wait(sidx, i, rows0_v, gsem0)
                scatter_add(didx, i, rows0_v)
                gather_start(sidx, i + 3, rows0_v, gsem0)
                gather_wait(sidx, i + 1, rows1_v, gsem1)
                scatter_add(didx, i + 1, rows1_v)
                gather_wait(sidx, i + 2, rows2_v, gsem2)
                scatter_add(didx, i + 2, rows2_v)
                gather_wait(sidx, i + 3, rows0_v, gsem0)
                scatter_add(didx, i + 3, rows0_v)

            plsc.subcore_barrier()
            slab_copy(acc_sh, o_hbm)

        @pl.when(c == 0)
        def _():
            run(xu_hbm, xi_hbm, 0, oi_hbm)

        @pl.when(c == 1)
        def _():
            run(xi_hbm, xu_hbm, 2, ou_hbm)

    return agg_kernel(x_user, x_item, idx)


def _mlp_body(
    hi_ref, hu_ref, w1ui_ref, b1ui_ref, w2ui_ref, b2ui_ref,
    w1iu_ref, b1iu_ref, w2iu_ref, b2iu_ref, oi_ref, ou_ref,
):
    hi = hi_ref[...]
    t = jnp.dot(hi, w1ui_ref[...]) + b1ui_ref[...]
    t = jnp.maximum(t, 0.0)
    oi_ref[...] = jnp.dot(t, w2ui_ref[...]) + b2ui_ref[...]
    hu = hu_ref[...]
    u = jnp.dot(hu, w1iu_ref[...]) + b1iu_ref[...]
    u = jnp.maximum(u, 0.0)
    ou_ref[...] = jnp.dot(u, w2iu_ref[...]) + b2iu_ref[...]


def _tc_mlps(hin_item, hin_user, w1ui, b1ui, w2ui, b2ui, w1iu, b1iu, w2iu, b2iu):
    n, d = hin_item.shape
    br = 2000
    assert n % br == 0
    spec_h = pl.BlockSpec((br, d), lambda i: (i, 0))
    spec_w = pl.BlockSpec((d, d), lambda i: (0, 0))
    spec_b = pl.BlockSpec((1, d), lambda i: (0, 0))
    return pl.pallas_call(
        _mlp_body,
        grid=(n // br,),
        in_specs=[spec_h, spec_h, spec_w, spec_b, spec_w, spec_b,
                  spec_w, spec_b, spec_w, spec_b],
        out_specs=[spec_h, spec_h],
        out_shape=[jax.ShapeDtypeStruct((n, d), jnp.float32)] * 2,
    )(
        hin_item, hin_user,
        w1ui, b1ui.reshape(1, d), w2ui, b2ui.reshape(1, d),
        w1iu, b1iu.reshape(1, d), w2iu, b2iu.reshape(1, d),
    )


def kernel(
    x_user, x_item, edge_index_user_item, edge_index_item_user,
    W1_ui, b1_ui, W2_ui, b2_ui, W1_iu, b1_iu, W2_iu, b2_iu,
):
    e = edge_index_user_item.shape[1]
    assert e % (_NSUB * _BATCH) == 0
    nc = e // (_NSUB * _BATCH)
    assert nc % _NPASS == 0
    cpp = nc // _NPASS
    idx = jnp.concatenate(
        [edge_index_user_item, edge_index_item_user]
    ).astype(jnp.int32).reshape(4, _NSUB, _NPASS, cpp, _BATCH)

    hin_item, hin_user = _sc_gin_aggregate(x_user, x_item, idx)
    h_item, h_user = _tc_mlps(
        hin_item, hin_user,
        W1_ui, b1_ui, W2_ui, b2_ui, W1_iu, b1_iu, W2_iu, b2_iu,
    )
    return (h_user, h_item)
